# baseline (speedup 1.0000x reference)
import functools

import jax
import jax.numpy as jnp
from jax import lax
from jax.experimental import pallas as pl
from jax.experimental.pallas import tpu as pltpu
from jax.experimental.pallas import tpu_sc as plsc

B, FN, FC, V, D, NCLS = 16384, 13, 26, 16, 16, 2
NC, NS, LANES = 2, 16, 16
NW = NC * NS
CH = B // NW
NBLK = CH // LANES

_TAB_OFF = 0
_WEMB_OFF = _TAB_OFF + FC * D * V
_WNUM_OFF = _WEMB_OFF + NCLS * FC * D
_BIAS_OFF = _WNUM_OFF + NCLS * FN * LANES
CONST_LEN = _BIAS_OFF + NCLS * LANES


def _tc_transpose_body(xcat_ref, xnum_ref, xcatt_ref, xnumt_ref):
    xcatt_ref[...] = xcat_ref[...].T
    xnumt_ref[...] = xnum_ref[...].T


def _sc_body(consts_hbm, xcatt_hbm, xnumt_hbm,
             out_hbm, consts_v, l_v, xcat_v, xnum_v, out_v, sem):
    cid = lax.axis_index("c")
    sid = lax.axis_index("s")
    wid = sid * NC + cid
    base = wid * CH

    cp_con = pltpu.async_copy(consts_hbm, consts_v, sem)
    cp_cat = pltpu.async_copy(xcatt_hbm.at[:, pl.ds(base, CH)], xcat_v, sem)
    cp_num = pltpu.async_copy(xnumt_hbm.at[:, pl.ds(base, CH)], xnum_v, sem)
    cp_con.wait()

    for i in range(FC):
        for c in range(NCLS):
            wvec = consts_v[pl.ds(_WEMB_OFF + (c * FC + i) * D, LANES)]
            acc = jnp.zeros((LANES,), jnp.float32)
            for d in range(D):
                acc = acc + consts_v[pl.ds(_TAB_OFF + i * (D * V) + d * V,
                                           LANES)] * wvec[d]
            l_v[pl.ds(c * (FC * V) + i * V, LANES)] = acc

    cp_cat.wait()
    cp_num.wait()

    iot = lax.iota(jnp.int32, LANES)

    def blk(j, carry):
        rb = j * LANES
        rows = rb + iot
        acc0 = consts_v[pl.ds(_BIAS_OFF, LANES)]
        acc1 = consts_v[pl.ds(_BIAS_OFF + LANES, LANES)]
        for n in range(FN):
            xv = xnum_v[n, pl.ds(rb, LANES)]
            acc0 = acc0 + xv * consts_v[pl.ds(_WNUM_OFF + n * LANES, LANES)]
            acc1 = acc1 + xv * consts_v[pl.ds(_WNUM_OFF + (FN + n) * LANES,
                                              LANES)]
        for i in range(FC):
            ci = xcat_v[i, pl.ds(rb, LANES)]
            acc0 = acc0 + plsc.load_gather(l_v, [ci + i * V])
            acc1 = acc1 + plsc.load_gather(l_v, [ci + (FC + i) * V])
        zc = jnp.zeros((LANES,), jnp.int32)
        plsc.store_scatter(out_v, [rows, zc], acc0)
        plsc.store_scatter(out_v, [rows, zc + 1], acc1)
        return carry

    lax.fori_loop(0, NBLK, blk, 0)

    pltpu.sync_copy(out_v, out_hbm.at[pl.ds(base, CH), :])


def kernel(x_num, x_cat, tables, W, b):
    x_cat_i = x_cat.astype(jnp.int32)
    tabt = tables.transpose(0, 2, 1).reshape(-1)
    wemb = W[:, FN:].reshape(-1)
    wnumb = jnp.broadcast_to(W[:, :FN][:, :, None],
                             (NCLS, FN, LANES)).reshape(-1)
    biasb = jnp.broadcast_to(b[:, None], (NCLS, LANES)).reshape(-1)
    consts = jnp.concatenate([tabt, wemb, wnumb, biasb])

    grid = 8
    rows = B // grid
    xcatt, xnumt = pl.pallas_call(
        _tc_transpose_body,
        grid=(grid,),
        in_specs=[
            pl.BlockSpec((rows, FC), lambda j: (j, 0)),
            pl.BlockSpec((rows, FN), lambda j: (j, 0)),
        ],
        out_specs=[
            pl.BlockSpec((FC, rows), lambda j: (0, j)),
            pl.BlockSpec((FN, rows), lambda j: (0, j)),
        ],
        out_shape=(jax.ShapeDtypeStruct((FC, B), jnp.int32),
                   jax.ShapeDtypeStruct((FN, B), jnp.float32)),
    )(x_cat_i, x_num)

    mesh = plsc.VectorSubcoreMesh(core_axis_name="c", subcore_axis_name="s")
    run = functools.partial(
        pl.kernel,
        mesh=mesh,
        compiler_params=pltpu.CompilerParams(needs_layout_passes=False,
                                             skip_device_barrier=True,
                                             use_tc_tiling_on_sc=True),
        out_type=jax.ShapeDtypeStruct((B, NCLS), jnp.float32),
        scratch_types=[
            pltpu.VMEM((CONST_LEN,), jnp.float32),
            pltpu.VMEM((NCLS * FC * V,), jnp.float32),
            pltpu.VMEM((FC, CH), jnp.int32),
            pltpu.VMEM((FN, CH), jnp.float32),
            pltpu.VMEM((CH, NCLS), jnp.float32),
            pltpu.SemaphoreType.DMA,
        ],
    )(_sc_body)
    return run(consts, xcatt, xnumt)


# speedup vs baseline: 1.0081x; 1.0081x over previous
import functools

import jax
import jax.numpy as jnp
from jax import lax
from jax.experimental import pallas as pl
from jax.experimental.pallas import tpu as pltpu
from jax.experimental.pallas import tpu_sc as plsc

B, FN, FC, V, D, NCLS = 16384, 13, 26, 16, 16, 2
NC, NS, LANES = 2, 16, 16
NW = NC * NS
CH = B // NW
NBLK = CH // LANES

_TAB_OFF = 0
_WEMB_OFF = _TAB_OFF + FC * D * V
_WNUM_OFF = _WEMB_OFF + NCLS * FC * D
_BIAS_OFF = _WNUM_OFF + NCLS * FN * LANES
CONST_LEN = _BIAS_OFF + NCLS * LANES


def _tc_body(xcat_ref, xnum_ref, eye_ref, w2_ref, xcatt_ref, denset_ref):
    xc = xcat_ref[...].astype(jnp.float32)
    xct = jax.lax.dot_general(eye_ref[...], xc, (((1,), (1,)), ((), ())),
                              preferred_element_type=jnp.float32)
    xcatt_ref[...] = xct.astype(jnp.int32)
    denset_ref[...] = jax.lax.dot_general(
        w2_ref[...], xnum_ref[...], (((1,), (1,)), ((), ())),
        preferred_element_type=jnp.float32)


def _sc_body(consts_hbm, xcatt_hbm, denset_hbm,
             out_hbm, consts_v, l_v, xcat_v, dense_v, out_v, sem):
    cid = lax.axis_index("c")
    sid = lax.axis_index("s")
    wid = sid * NC + cid
    base = wid * CH

    cp_con = pltpu.async_copy(consts_hbm, consts_v, sem)
    cp_cat = pltpu.async_copy(xcatt_hbm.at[:, pl.ds(base, CH)], xcat_v, sem)
    cp_den = pltpu.async_copy(denset_hbm.at[:, pl.ds(base, CH)], dense_v, sem)
    cp_con.wait()

    for i in range(FC):
        for c in range(NCLS):
            wvec = consts_v[pl.ds(_WEMB_OFF + (c * FC + i) * D, LANES)]
            acc = jnp.zeros((LANES,), jnp.float32)
            for d in range(D):
                acc = acc + consts_v[pl.ds(_TAB_OFF + i * (D * V) + d * V,
                                           LANES)] * wvec[d]
            l_v[pl.ds(c * (FC * V) + i * V, LANES)] = acc

    cp_cat.wait()
    cp_den.wait()

    iot = lax.iota(jnp.int32, LANES)

    def blk(j, carry):
        rb = j * LANES
        rows = rb + iot
        acc0 = consts_v[pl.ds(_BIAS_OFF, LANES)] + dense_v[0, pl.ds(rb, LANES)]
        acc1 = consts_v[pl.ds(_BIAS_OFF + LANES, LANES)] + dense_v[
            1, pl.ds(rb, LANES)]
        for i in range(FC):
            ci = xcat_v[i, pl.ds(rb, LANES)]
            acc0 = acc0 + plsc.load_gather(l_v, [ci + i * V])
            acc1 = acc1 + plsc.load_gather(l_v, [ci + (FC + i) * V])
        zc = jnp.zeros((LANES,), jnp.int32)
        plsc.store_scatter(out_v, [rows, zc], acc0)
        plsc.store_scatter(out_v, [rows, zc + 1], acc1)
        return carry

    lax.fori_loop(0, NBLK, blk, 0)

    pltpu.sync_copy(out_v, out_hbm.at[pl.ds(base, CH), :])


def kernel(x_num, x_cat, tables, W, b):
    x_cat_i = x_cat.astype(jnp.int32)
    tabt = tables.transpose(0, 2, 1).reshape(-1)
    wemb = W[:, FN:].reshape(-1)
    wnumb = jnp.broadcast_to(W[:, :FN][:, :, None],
                             (NCLS, FN, LANES)).reshape(-1)
    biasb = jnp.broadcast_to(b[:, None], (NCLS, LANES)).reshape(-1)
    consts = jnp.concatenate([tabt, wemb, wnumb, biasb])

    grid = 8
    rows = B // grid
    eye = jnp.eye(FC, dtype=jnp.float32)
    w2 = W[:, :FN]
    xcatt, denset = pl.pallas_call(
        _tc_body,
        grid=(grid,),
        in_specs=[
            pl.BlockSpec((rows, FC), lambda j: (j, 0)),
            pl.BlockSpec((rows, FN), lambda j: (j, 0)),
            pl.BlockSpec((FC, FC), lambda j: (0, 0)),
            pl.BlockSpec((NCLS, FN), lambda j: (0, 0)),
        ],
        out_specs=[
            pl.BlockSpec((FC, rows), lambda j: (0, j)),
            pl.BlockSpec((NCLS, rows), lambda j: (0, j)),
        ],
        out_shape=(jax.ShapeDtypeStruct((FC, B), jnp.int32),
                   jax.ShapeDtypeStruct((NCLS, B), jnp.float32)),
    )(x_cat_i, x_num, eye, w2)

    mesh = plsc.VectorSubcoreMesh(core_axis_name="c", subcore_axis_name="s")
    run = functools.partial(
        pl.kernel,
        mesh=mesh,
        compiler_params=pltpu.CompilerParams(needs_layout_passes=False,
                                             skip_device_barrier=True,
                                             use_tc_tiling_on_sc=True),
        out_type=jax.ShapeDtypeStruct((B, NCLS), jnp.float32),
        scratch_types=[
            pltpu.VMEM((CONST_LEN,), jnp.float32),
            pltpu.VMEM((NCLS * FC * V,), jnp.float32),
            pltpu.VMEM((FC, CH), jnp.int32),
            pltpu.VMEM((NCLS, CH), jnp.float32),
            pltpu.VMEM((CH, NCLS), jnp.float32),
            pltpu.SemaphoreType.DMA,
        ],
    )(_sc_body)
    return run(consts, xcatt, denset)


# speedup vs baseline: 1.0670x; 1.0584x over previous
import functools

import jax
import jax.numpy as jnp
from jax import lax
from jax.experimental import pallas as pl
from jax.experimental.pallas import tpu as pltpu
from jax.experimental.pallas import tpu_sc as plsc

B, FN, FC, V, D, NCLS = 16384, 13, 26, 16, 16, 2
NC, NS, LANES = 2, 16, 16
NW = NC * NS
CH = B // NW
CHK = 128
NCHK = CH // CHK
NBLK = CHK // LANES
TSTRIDE = CHK + 1

_TAB_OFF = 0
_WEMB_OFF = _TAB_OFF + FC * D * V
_WNUM_OFF = _WEMB_OFF + NCLS * FC * D
_BIAS_OFF = _WNUM_OFF + NCLS * FN * LANES
CONST_LEN = _BIAS_OFF + NCLS * LANES


def _sc_body(consts_hbm, xcat_hbm, xnum_hbm, out_hbm,
             consts_v, l_v, xcatt_v, xnumt_v,
             xcat0, xcat1, xnum0, xnum1, out0, out1,
             csem, isem0, isem1, osem0, osem1):
    cid = lax.axis_index("c")
    sid = lax.axis_index("s")
    wid = sid * NC + cid
    base = wid * CH

    xcats = [xcat0, xcat1]
    xnums = [xnum0, xnum1]
    outs = [out0, out1]
    isems = [isem0, isem1]
    osems = [osem0, osem1]

    cp_con = pltpu.async_copy(consts_hbm, consts_v, csem)

    def start_in(k):
        s = k & 1
        r0 = base + k * CHK
        return (
            pltpu.async_copy(xcat_hbm.at[pl.ds(r0, CHK), :], xcats[s],
                             isems[s]),
            pltpu.async_copy(xnum_hbm.at[pl.ds(r0, CHK), :], xnums[s],
                             isems[s]),
        )

    pend = {0: start_in(0)}

    cp_con.wait()

    for i in range(FC):
        for c in range(NCLS):
            wvec = consts_v[pl.ds(_WEMB_OFF + (c * FC + i) * D, LANES)]
            acc = jnp.zeros((LANES,), jnp.float32)
            for d in range(D):
                acc = acc + consts_v[pl.ds(_TAB_OFF + i * (D * V) + d * V,
                                           LANES)] * wvec[d]
            l_v[pl.ds(c * (FC * V) + i * V, LANES)] = acc

    iot = lax.iota(jnp.int32, LANES)
    tidx_a = iot * TSTRIDE
    tidx_b = (iot + 10) * TSTRIDE
    tidx_n = iot * TSTRIDE
    ow = {}
    for k in range(NCHK):
        s = k & 1
        if k + 1 < NCHK:
            pend[k + 1] = start_in(k + 1)
        for cp in pend.pop(k):
            cp.wait()
        if k >= 2:
            ow.pop(k - 2).wait()

        xcat_v, xnum_v, out_v = xcats[s], xnums[s], outs[s]

        def trow(j, carry):
            rb = j * 4
            for u in range(4):
                r = rb + u
                fa = xcat_v[r, pl.ds(0, LANES)]
                fb = xcat_v[r, pl.ds(FC - LANES, LANES)]
                plsc.store_scatter(xcatt_v, [tidx_a + r], fa)
                plsc.store_scatter(xcatt_v, [tidx_b + r], fb)
                fn = xnum_v[r, pl.ds(0, LANES)]
                plsc.store_scatter(xnumt_v, [tidx_n + r], fn)
            return carry

        lax.fori_loop(0, CHK // 4, trow, 0)

        def blk(j, carry):
            rb = j * LANES
            rows = rb + iot
            acc0 = consts_v[pl.ds(_BIAS_OFF, LANES)]
            acc1 = consts_v[pl.ds(_BIAS_OFF + LANES, LANES)]
            for n in range(FN):
                xv = xnumt_v[pl.ds(n * TSTRIDE + rb, LANES)]
                acc0 = acc0 + xv * consts_v[pl.ds(_WNUM_OFF + n * LANES,
                                                  LANES)]
                acc1 = acc1 + xv * consts_v[pl.ds(_WNUM_OFF + (FN + n) * LANES,
                                                  LANES)]
            for i in range(FC):
                ci = xcatt_v[pl.ds(i * TSTRIDE + rb, LANES)]
                acc0 = acc0 + plsc.load_gather(l_v, [ci + i * V])
                acc1 = acc1 + plsc.load_gather(l_v, [ci + (FC + i) * V])
            zc = jnp.zeros((LANES,), jnp.int32)
            plsc.store_scatter(out_v, [rows, zc], acc0)
            plsc.store_scatter(out_v, [rows, zc + 1], acc1)
            return carry

        lax.fori_loop(0, NBLK, blk, 0)
        ow[k] = pltpu.async_copy(
            out_v, out_hbm.at[pl.ds(base + k * CHK, CHK), :], osems[s])

    for k in sorted(ow):
        ow.pop(k).wait()


def kernel(x_num, x_cat, tables, W, b):
    x_cat_i = x_cat.astype(jnp.int32)
    tabt = tables.transpose(0, 2, 1).reshape(-1)
    wemb = W[:, FN:].reshape(-1)
    wnumb = jnp.broadcast_to(W[:, :FN][:, :, None],
                             (NCLS, FN, LANES)).reshape(-1)
    biasb = jnp.broadcast_to(b[:, None], (NCLS, LANES)).reshape(-1)
    consts = jnp.concatenate([tabt, wemb, wnumb, biasb])

    mesh = plsc.VectorSubcoreMesh(core_axis_name="c", subcore_axis_name="s")
    run = functools.partial(
        pl.kernel,
        mesh=mesh,
        compiler_params=pltpu.CompilerParams(needs_layout_passes=False,
                                             skip_device_barrier=True,
                                             use_tc_tiling_on_sc=True),
        out_type=jax.ShapeDtypeStruct((B, NCLS), jnp.float32),
        scratch_types=[
            pltpu.VMEM((CONST_LEN,), jnp.float32),
            pltpu.VMEM((NCLS * FC * V,), jnp.float32),
            pltpu.VMEM((FC * TSTRIDE,), jnp.int32),
            pltpu.VMEM((LANES * TSTRIDE,), jnp.float32),
            pltpu.VMEM((CHK, FC), jnp.int32),
            pltpu.VMEM((CHK, FC), jnp.int32),
            pltpu.VMEM((CHK, FN), jnp.float32),
            pltpu.VMEM((CHK, FN), jnp.float32),
            pltpu.VMEM((CHK, NCLS), jnp.float32),
            pltpu.VMEM((CHK, NCLS), jnp.float32),
            pltpu.SemaphoreType.DMA,
            pltpu.SemaphoreType.DMA,
            pltpu.SemaphoreType.DMA,
            pltpu.SemaphoreType.DMA,
            pltpu.SemaphoreType.DMA,
        ],
    )(_sc_body)
    return run(consts, x_cat_i, x_num)
